# baseline (speedup 1.0000x reference)
import functools

import jax
import jax.numpy as jnp
from jax import lax
from jax.experimental import pallas as pl
from jax.experimental.pallas import tpu as pltpu
from jax.experimental.pallas import tpu_sc as plsc

NUM_EMB = 1000000
D = 64
B_TOK = 4096
T_TOK = 200
B = B_TOK * T_TOK
NC = 2
NS = 16
NW = NC * NS
CHUNK = 128
PER_W = B // NW
NCHUNK = PER_W // CHUNK


def _sc_gather(table, idx3):
    mesh = plsc.VectorSubcoreMesh(core_axis_name="c", subcore_axis_name="s")

    @functools.partial(
        pl.kernel,
        mesh=mesh,
        out_type=jax.ShapeDtypeStruct((B, D), jnp.float32),
        compiler_params=pltpu.CompilerParams(use_tc_tiling_on_sc=False),
        scratch_types=[
            pltpu.VMEM((NCHUNK, CHUNK), jnp.int32),
            pltpu.VMEM((CHUNK, D), jnp.float32),
            pltpu.SemaphoreType.DMA,
        ],
    )
    def k(table_hbm, idx_hbm, out_hbm, idx_v, rows_v, sem):
        wid = lax.axis_index("s") * NC + lax.axis_index("c")
        base = wid * PER_W
        pltpu.sync_copy(idx_hbm.at[wid], idx_v)

        def body(j, carry):
            pltpu.async_copy(table_hbm.at[idx_v.at[j]], rows_v, sem).wait()
            pltpu.sync_copy(rows_v, out_hbm.at[pl.ds(base + j * CHUNK, CHUNK)])
            return carry

        lax.fori_loop(0, NCHUNK, body, 0)

    return k(table, idx3)


def kernel(token_ids, embeddings):
    idx3 = token_ids.reshape(NW, NCHUNK, CHUNK)
    out = _sc_gather(embeddings, idx3)
    return out.reshape(B_TOK, T_TOK, D)


# speedup vs baseline: 1.1144x; 1.1144x over previous
import functools

import jax
import jax.numpy as jnp
from jax import lax
from jax.experimental import pallas as pl
from jax.experimental.pallas import tpu as pltpu
from jax.experimental.pallas import tpu_sc as plsc

NUM_EMB = 1000000
D = 64
B_TOK = 4096
T_TOK = 200
B = B_TOK * T_TOK
NC = 2
NS = 16
NW = NC * NS
CHUNK = 128
PER_W = B // NW
NCHUNK = PER_W // CHUNK
K = 5
GROUPS = NCHUNK // K
GROW = K * CHUNK


def _sc_gather(table, idx3):
    mesh = plsc.VectorSubcoreMesh(core_axis_name="c", subcore_axis_name="s")

    @functools.partial(
        pl.kernel,
        mesh=mesh,
        out_type=jax.ShapeDtypeStruct((B, D), jnp.float32),
        compiler_params=pltpu.CompilerParams(use_tc_tiling_on_sc=False),
        scratch_types=[
            pltpu.VMEM((NCHUNK, CHUNK), jnp.int32),
            pltpu.VMEM((2 * GROW, D), jnp.float32),
            pltpu.SemaphoreType.DMA,
            pltpu.SemaphoreType.DMA,
        ],
    )
    def k(table_hbm, idx_hbm, out_hbm, idx_v, rows_v, sem_in, sem_out):
        wid = lax.axis_index("s") * NC + lax.axis_index("c")
        base = wid * PER_W
        pltpu.sync_copy(idx_hbm.at[wid], idx_v)

        def fire_gathers(g, half):
            for b in range(K):
                pltpu.async_copy(
                    table_hbm.at[idx_v.at[g * K + b]],
                    rows_v.at[pl.ds(half * GROW + b * CHUNK, CHUNK)],
                    sem_in,
                )

        def drain(sem, half):
            pltpu.make_async_copy(
                out_hbm.at[pl.ds(base, GROW)],
                rows_v.at[pl.ds(half * GROW, GROW)],
                sem,
            ).wait()

        fire_gathers(0, 0)

        def body(g, carry):
            half = lax.rem(g, 2)

            @pl.when(g >= 1)
            def _():
                drain(sem_out, 1 - half)

            @pl.when(g + 1 < GROUPS)
            def _():
                fire_gathers(g + 1, 1 - half)

            drain(sem_in, half)
            pltpu.async_copy(
                rows_v.at[pl.ds(half * GROW, GROW)],
                out_hbm.at[pl.ds(base + g * GROW, GROW)],
                sem_out,
            )
            return carry

        lax.fori_loop(0, GROUPS, body, 0)
        drain(sem_out, (GROUPS - 1) % 2)

    return k(table, idx3)


def kernel(token_ids, embeddings):
    idx3 = token_ids.reshape(NW, NCHUNK, CHUNK)
    out = _sc_gather(embeddings, idx3)
    return out.reshape(B_TOK, T_TOK, D)


# speedup vs baseline: 1.3615x; 1.2217x over previous
import functools

import jax
import jax.numpy as jnp
from jax import lax
from jax.experimental import pallas as pl
from jax.experimental.pallas import tpu as pltpu
from jax.experimental.pallas import tpu_sc as plsc

NUM_EMB = 1000000
D = 64
DP = 128
B_TOK = 4096
T_TOK = 200
TP = 256
NC = 2
NS = 16
NW = NC * NS
BPW = B_TOK // NW
CHUNK = 128
REM = T_TOK - CHUNK


def _sc_gather(table, idx3):
    mesh = plsc.VectorSubcoreMesh(core_axis_name="c", subcore_axis_name="s")

    @functools.partial(
        pl.kernel,
        mesh=mesh,
        out_type=jax.ShapeDtypeStruct((B_TOK, T_TOK, DP), jnp.float32),
        compiler_params=pltpu.CompilerParams(use_tc_tiling_on_sc=True),
        scratch_types=[
            pltpu.VMEM((BPW, TP), jnp.int32),
            pltpu.VMEM((2 * T_TOK, DP), jnp.float32),
            pltpu.SemaphoreType.DMA,
            pltpu.SemaphoreType.DMA,
        ],
    )
    def k(table_hbm, idx_hbm, out_hbm, idx_v, rows_v, sem_in, sem_out):
        wid = lax.axis_index("s") * NC + lax.axis_index("c")
        pltpu.sync_copy(idx_hbm.at[wid], idx_v)

        def gathers(j, half):
            return (
                pltpu.make_async_copy(
                    table_hbm.at[idx_v.at[j, pl.ds(0, CHUNK)]],
                    rows_v.at[pl.ds(half * T_TOK, CHUNK)],
                    sem_in,
                ),
                pltpu.make_async_copy(
                    table_hbm.at[idx_v.at[j, pl.ds(CHUNK, REM)]],
                    rows_v.at[pl.ds(half * T_TOK + CHUNK, REM)],
                    sem_in,
                ),
            )

        def out_copy(j, half):
            return pltpu.make_async_copy(
                rows_v.at[pl.ds(half * T_TOK, T_TOK)],
                out_hbm.at[wid * BPW + j],
                sem_out,
            )

        for c in gathers(0, 0):
            c.start()

        def body(j, carry):
            half = lax.rem(j, 2)

            @pl.when(j >= 1)
            def _():
                out_copy(j - 1, 1 - half).wait()

            @pl.when(j + 1 < BPW)
            def _():
                for c in gathers(j + 1, 1 - half):
                    c.start()

            for c in gathers(j, half):
                c.wait()
            out_copy(j, half).start()
            return carry

        lax.fori_loop(0, BPW, body, 0)
        out_copy(BPW - 1, (BPW - 1) % 2).wait()

    return k(table, idx3)


def kernel(token_ids, embeddings):
    table = jnp.pad(embeddings, ((0, 0), (0, DP - D)))
    idx3 = jnp.pad(token_ids.reshape(NW, BPW, T_TOK), ((0, 0), (0, 0), (0, TP - T_TOK)))
    out = _sc_gather(table, idx3)
    return out[:, :, :D]
